# baseline (speedup 1.0000x reference)
import functools
import math

import jax
import jax.numpy as jnp
from jax import lax
from jax.experimental import pallas as pl
from jax.experimental.pallas import tpu as pltpu
from jax.experimental.pallas import tpu_sc as plsc

_NC = 2
_NS = 16
_NQ = 4
_QW = 16
_LANES = 128
_K = 16

_BN = 1024


def _ceil_to(x, m):
  return (x + m - 1) // m * m


def _make_segsum(n_pad, rows_total):
  rt = rows_total // _NS
  groups = rt // _K
  t_rows = n_pad // _NS
  io_rows = t_rows // 7
  mesh = plsc.VectorSubcoreMesh(
      core_axis_name="c", subcore_axis_name="s",
      num_cores=_NC, num_subcores=_NS)

  @functools.partial(
      pl.kernel,
      out_type=jax.ShapeDtypeStruct((_NQ * n_pad, _QW), jnp.float32),
      mesh=mesh,
      scratch_types=[
          pltpu.VMEM((2, _K, 2, _LANES), jnp.int32),
          pltpu.VMEM((2, _K, _LANES, _QW), jnp.float32),
          pltpu.VMEM((io_rows, _QW), jnp.float32),
          pltpu.VMEM_SHARED((n_pad, _QW), jnp.float32),
          pltpu.SemaphoreType.DMA,
          pltpu.SemaphoreType.DMA,
          pltpu.SemaphoreType.DMA,
          pltpu.SemaphoreType.DMA,
      ],
      compiler_params=pltpu.CompilerParams(use_tc_tiling_on_sc=False),
  )
  def segsum(feat4, comb4, zeros_hbm, out, idx_v, rows_v, io_v,
             acc, gsem0, gsem1, ssem0, ssem1):
    c = lax.axis_index("c")
    s = lax.axis_index("s")
    gsem = (gsem0, gsem1)
    ssem = (ssem0, ssem1)
    zdrain = zeros_hbm.at[pl.ds(0, _LANES)]
    t0 = s * t_rows
    tile_row0 = s * rt

    def drain_scatters(b):
      for j in range(_K):
        pltpu.make_async_copy(zdrain, rows_v.at[b].at[j], ssem[b]).wait()

    for q_local in range(2):
      q = c * 2 + q_local
      pltpu.sync_copy(zeros_hbm, io_v)
      for j in range(7):
        pltpu.sync_copy(io_v, acc.at[pl.ds(t0 + j * io_rows, io_rows)])
      plsc.subcore_barrier()

      def outer(go, carry):
        for b in range(2):
          g = 2 * go + b
          r0 = tile_row0 + g * _K

          @pl.when(go > 0)
          def _():
            drain_scatters(b)

          pltpu.sync_copy(comb4.at[pl.ds(q * rows_total + r0, _K)],
                          idx_v.at[b])
          handles = [
              pltpu.async_copy(feat4.at[idx_v.at[b].at[j, 0]],
                               rows_v.at[b].at[j], gsem[b])
              for j in range(_K)
          ]
          for h in handles:
            h.wait()
          for j in range(_K):
            pltpu.async_copy(rows_v.at[b].at[j],
                             acc.at[idx_v.at[b].at[j, 1]],
                             ssem[b], add=True)
        return carry

      lax.fori_loop(0, groups // 2, outer, 0)
      for b in range(2):
        drain_scatters(b)
      plsc.subcore_barrier()
      for j in range(7):
        pltpu.sync_copy(acc.at[pl.ds(t0 + j * io_rows, io_rows)], io_v)
        pltpu.sync_copy(
            io_v, out.at[pl.ds(q * n_pad + t0 + j * io_rows, io_rows)])
      plsc.subcore_barrier()

  return segsum


def _make_degree(n_pad, rows_total):
  rt = rows_total // (_NC * _NS)
  groups = rt // _K
  t_rows = n_pad // _NS
  io_rows = t_rows // 7
  mesh = plsc.VectorSubcoreMesh(
      core_axis_name="c", subcore_axis_name="s",
      num_cores=_NC, num_subcores=_NS)

  @functools.partial(
      pl.kernel,
      out_type=jax.ShapeDtypeStruct((_NC * n_pad, _QW), jnp.float32),
      mesh=mesh,
      scratch_types=[
          pltpu.VMEM((rt, _LANES), jnp.int32),
          pltpu.VMEM((_LANES, _QW), jnp.float32),
          pltpu.VMEM((io_rows, _QW), jnp.float32),
          pltpu.VMEM_SHARED((n_pad, _QW), jnp.float32),
          pltpu.SemaphoreType.DMA,
      ],
      compiler_params=pltpu.CompilerParams(use_tc_tiling_on_sc=False),
  )
  def degree(dst, ones_hbm, zeros_hbm, out, dst_v, ones_v, io_v, acc, sem):
    c = lax.axis_index("c")
    s = lax.axis_index("s")
    pltpu.sync_copy(ones_hbm, ones_v)
    pltpu.sync_copy(zeros_hbm, io_v)
    t0 = s * t_rows
    for j in range(7):
      pltpu.sync_copy(io_v, acc.at[pl.ds(t0 + j * io_rows, io_rows)])
    plsc.subcore_barrier()

    wid = c * _NS + s
    pltpu.sync_copy(dst.at[pl.ds(wid * rt, rt)], dst_v)

    def group(go, carry):
      for j in range(_K):
        pltpu.async_copy(ones_v, acc.at[dst_v.at[go * _K + j]], sem,
                         add=True)
      return carry

    lax.fori_loop(0, groups, group, 0)

    def drain(go, carry):
      for j in range(_K):
        pltpu.make_async_copy(zeros_hbm.at[pl.ds(0, _LANES)], ones_v,
                              sem).wait()
      return carry

    lax.fori_loop(0, groups, drain, 0)
    plsc.subcore_barrier()
    for j in range(7):
      pltpu.sync_copy(acc.at[pl.ds(t0 + j * io_rows, io_rows)], io_v)
      pltpu.sync_copy(
          io_v, out.at[pl.ds(c * n_pad + t0 + j * io_rows, io_rows)])

  return degree


def _featgen(x_pad, w_pad, b, n_pad):
  nb = n_pad // _BN

  def body(x_ref, w_ref, b_ref, o_ref):
    y = jnp.dot(x_ref[...], w_ref[...], preferred_element_type=jnp.float32)
    y = jnp.maximum(y + b_ref[...], 0.0)
    for q in range(_NQ):
      o_ref[q] = y[:, q * _QW:(q + 1) * _QW]

  return pl.pallas_call(
      body,
      grid=(nb,),
      in_specs=[
          pl.BlockSpec((_BN, 8), lambda i: (i, 0)),
          pl.BlockSpec((8, 64), lambda i: (0, 0)),
          pl.BlockSpec((1, 64), lambda i: (0, 0)),
      ],
      out_specs=pl.BlockSpec((_NQ, _BN, _QW), lambda i: (0, i, 0)),
      out_shape=jax.ShapeDtypeStruct((_NQ, n_pad, _QW), jnp.float32),
  )(x_pad, w_pad, b)


def _router_update(fr, st, sc, deg8, wr, br, n_pad):
  nb = n_pad // _BN

  def body(fr_ref, st_ref, sc_ref, deg_ref, w_ref, b_ref, o_ref):
    inv = 1.0 / jnp.maximum(deg_ref[:, 0:1], 1.0)
    y = b_ref[...]
    for q in range(_NQ):
      y = y + jnp.dot(st_ref[q] * inv, w_ref[q * _QW:(q + 1) * _QW, :],
                      preferred_element_type=jnp.float32)
      y = y + jnp.dot(sc_ref[q], w_ref[64 + q * _QW:64 + (q + 1) * _QW, :],
                      preferred_element_type=jnp.float32)
    y = jnp.maximum(y, 0.0)
    for q in range(_NQ):
      o_ref[q] = fr_ref[q] + y[:, q * _QW:(q + 1) * _QW]

  blk = pl.BlockSpec((_NQ, _BN, _QW), lambda i: (0, i, 0))
  return pl.pallas_call(
      body,
      grid=(nb,),
      in_specs=[
          blk, blk, blk,
          pl.BlockSpec((_BN, _QW), lambda i: (i, 0)),
          pl.BlockSpec((128, 64), lambda i: (0, 0)),
          pl.BlockSpec((1, 64), lambda i: (0, 0)),
      ],
      out_specs=blk,
      out_shape=jax.ShapeDtypeStruct((_NQ, n_pad, _QW), jnp.float32),
  )(fr, st, sc, deg8, wr, br)


def _packet_update(fp, sp, wp, bp, n_pad):
  nb = n_pad // _BN

  def body(fp_ref, sp_ref, w_ref, b_ref, o_ref):
    y = b_ref[...]
    for q in range(_NQ):
      y = y + jnp.dot(sp_ref[q], w_ref[q * _QW:(q + 1) * _QW, :],
                      preferred_element_type=jnp.float32)
    y = jnp.maximum(y, 0.0)
    for q in range(_NQ):
      o_ref[q] = fp_ref[q] + y[:, q * _QW:(q + 1) * _QW]

  blk = pl.BlockSpec((_NQ, _BN, _QW), lambda i: (0, i, 0))
  return pl.pallas_call(
      body,
      grid=(nb,),
      in_specs=[
          blk, blk,
          pl.BlockSpec((64, 64), lambda i: (0, 0)),
          pl.BlockSpec((1, 64), lambda i: (0, 0)),
      ],
      out_specs=blk,
      out_shape=jax.ShapeDtypeStruct((_NQ, n_pad, _QW), jnp.float32),
  )(fp, sp, wp, bp)


def _readout(fr, fp, n_r, n_p, w1, b1, w2, b2, w3p, b3p, n_pad):
  nb = n_pad // _BN

  def body(fr_ref, fp_ref, w1_ref, b1_ref, w2_ref, b2_ref, w3_ref, b3_ref,
           o_ref, acc_ref):
    i = pl.program_id(0)

    @pl.when(i == 0)
    def _():
      acc_ref[...] = jnp.zeros_like(acc_ref)

    rows = i * _BN + lax.broadcasted_iota(jnp.int32, (_BN, 1), 0)
    mp = jnp.where(rows < n_p, 1.0, 0.0)
    mr = jnp.where(rows < n_r, 1.0, 0.0)
    for q in range(_NQ):
      acc_ref[:, q * _QW:(q + 1) * _QW] += jnp.sum(
          fp_ref[q] * mp, axis=0, keepdims=True)
      acc_ref[:, 64 + q * _QW:64 + (q + 1) * _QW] += jnp.sum(
          fr_ref[q] * mr, axis=0, keepdims=True)

    @pl.when(i == nb - 1)
    def _():
      scale = jnp.concatenate(
          [jnp.full((1, 64), 1.0 / n_p, jnp.float32),
           jnp.full((1, 64), 1.0 / n_r, jnp.float32)], axis=1)
      emb = acc_ref[...] * scale
      h = jnp.maximum(
          jnp.dot(emb, w1_ref[...], preferred_element_type=jnp.float32)
          + b1_ref[...], 0.0)
      h = jnp.maximum(
          jnp.dot(h, w2_ref[...], preferred_element_type=jnp.float32)
          + b2_ref[...], 0.0)
      y = jnp.dot(h, w3_ref[...], preferred_element_type=jnp.float32) \
          + b3_ref[...]
      o_ref[...] = jnp.broadcast_to(y, (8, 128))

  blk = pl.BlockSpec((_NQ, _BN, _QW), lambda i: (0, i, 0))
  full = lambda r, c: pl.BlockSpec((r, c), lambda i: (0, 0))
  return pl.pallas_call(
      body,
      grid=(nb,),
      in_specs=[
          blk, blk,
          full(128, 64), full(1, 64),
          full(64, 64), full(1, 64),
          full(64, 128), full(1, 128),
      ],
      out_specs=pl.BlockSpec((8, 128), lambda i: (0, 0)),
      out_shape=jax.ShapeDtypeStruct((8, 128), jnp.float32),
      scratch_shapes=[pltpu.VMEM((1, 128), jnp.float32)],
  )(fr, fp, w1, b1, w2, b2, w3p, b3p)


def kernel(router_embed, packet_embed, pass_edge_index, transfer_edge_index,
           connect_edge_index, W_node, b_node, W_hyper, b_hyper,
           c1_Wr, c1_br, c1_Wp, c1_bp, c2_Wr, c2_br, c2_Wp, c2_bp,
           h_W1, h_b1, h_W2, h_b2, h_W3, h_b3):
  n_r = router_embed.shape[0]
  n_p = packet_embed.shape[0]
  n = max(n_r, n_p)
  n_pad = _ceil_to(n + 1, math.lcm(_NS * 7 * 8, _BN))
  e = pass_edge_index.shape[1]
  rows_total = _ceil_to((e + _LANES - 1) // _LANES, 2 * _NS * _K)
  e_pad = rows_total * _LANES

  def prep_edges(ei):
    src = ei[0].astype(jnp.int32)
    dst = ei[1].astype(jnp.int32)
    src = jnp.pad(src, (0, e_pad - e)).reshape(rows_total, 1, _LANES)
    dst = jnp.pad(dst, (0, e_pad - e), constant_values=n)
    dst = dst.reshape(rows_total, 1, _LANES)
    comb4 = jnp.concatenate(
        [jnp.concatenate([src + q * n_pad, dst], axis=1)
         for q in range(_NQ)], axis=0)
    return comb4, dst.reshape(rows_total, _LANES)

  pass_comb4, pass_dst = prep_edges(pass_edge_index)
  tr_comb4, tr_dst = prep_edges(transfer_edge_index)
  co_comb4, co_dst = prep_edges(connect_edge_index)

  segsum = _make_segsum(n_pad, rows_total)
  degree = _make_degree(n_pad, rows_total)

  zeros16 = jnp.zeros((n_pad // _NS // 7, _QW), jnp.float32)
  ones16 = jnp.ones((_LANES, _QW), jnp.float32)

  deg2 = degree(tr_dst, ones16, zeros16)
  deg8 = deg2[:n_pad] + deg2[n_pad:]

  re_pad = jnp.pad(router_embed, ((0, n_pad - n_r), (0, 8 - 5)))
  pe_pad = jnp.pad(packet_embed, ((0, n_pad - n_p), (0, 8 - 2)))
  wn_pad = jnp.pad(W_node, ((0, 8 - 5), (0, 0)))
  wh_pad = jnp.pad(W_hyper, ((0, 8 - 2), (0, 0)))
  fr = _featgen(re_pad, wn_pad, b_node.reshape(1, 64), n_pad)
  fp = _featgen(pe_pad, wh_pad, b_hyper.reshape(1, 64), n_pad)

  for wr, br, wp, bp in ((c1_Wr, c1_br, c1_Wp, c1_bp),
                         (c2_Wr, c2_br, c2_Wp, c2_bp)):
    fr4 = fr.reshape(_NQ * n_pad, _QW)
    fp4 = fp.reshape(_NQ * n_pad, _QW)
    st = segsum(fp4, tr_comb4, zeros16).reshape(_NQ, n_pad, _QW)
    sc = segsum(fr4, co_comb4, zeros16).reshape(_NQ, n_pad, _QW)
    sp = segsum(fr4, pass_comb4, zeros16).reshape(_NQ, n_pad, _QW)
    fr_new = _router_update(fr, st, sc, deg8, wr, br.reshape(1, 64), n_pad)
    fp_new = _packet_update(fp, sp, wp, bp.reshape(1, 64), n_pad)
    fr, fp = fr_new, fp_new

  w3p = jnp.pad(h_W3, ((0, 0), (0, 128 - 2)))
  b3p = jnp.pad(h_b3, (0, 128 - 2)).reshape(1, 128)
  out = _readout(fr, fp, n_r, n_p, h_W1, h_b1.reshape(1, 64),
                 h_W2, h_b2.reshape(1, 64), w3p, b3p, n_pad)
  return out[0:1, 0:2]


# speedup vs baseline: 1.6903x; 1.6903x over previous
import functools
import math

import jax
import jax.numpy as jnp
from jax import lax
from jax.experimental import pallas as pl
from jax.experimental.pallas import tpu as pltpu
from jax.experimental.pallas import tpu_sc as plsc

_NC = 2
_NS = 16
_NQ = 4
_QW = 16
_LANES = 128
_K = 8

_BN = 1024


def _ceil_to(x, m):
  return (x + m - 1) // m * m


def _make_segsum(n_pad, rows_total):
  rt = rows_total // _NS
  groups = rt // _K
  t_rows = n_pad // _NS
  io_rows = t_rows // 7
  mesh = plsc.VectorSubcoreMesh(
      core_axis_name="c", subcore_axis_name="s",
      num_cores=_NC, num_subcores=_NS)

  @functools.partial(
      pl.kernel,
      out_type=jax.ShapeDtypeStruct((_NQ * n_pad, _QW), jnp.float32),
      mesh=mesh,
      scratch_types=[
          pltpu.VMEM((2, _K, 2, _LANES), jnp.int32),
          pltpu.VMEM((2, _K, _LANES, _QW), jnp.float32),
          pltpu.VMEM((io_rows, _QW), jnp.float32),
          pltpu.VMEM_SHARED((n_pad, _QW), jnp.float32),
          pltpu.SemaphoreType.DMA,
          pltpu.SemaphoreType.DMA,
          pltpu.SemaphoreType.DMA,
          pltpu.SemaphoreType.DMA,
      ],
      compiler_params=pltpu.CompilerParams(use_tc_tiling_on_sc=False),
  )
  def segsum(feat4, comb4, zeros_hbm, out, idx_v, rows_v, io_v,
             acc, gsem0, gsem1, ssem0, ssem1):
    c = lax.axis_index("c")
    s = lax.axis_index("s")
    gsem = (gsem0, gsem1)
    ssem = (ssem0, ssem1)
    zdrain = zeros_hbm.at[pl.ds(0, _LANES)]
    t0 = s * t_rows
    tile_row0 = s * rt

    for q_local in range(2):
      q = c * 2 + q_local
      pltpu.sync_copy(zeros_hbm, io_v)
      for j in range(7):
        pltpu.sync_copy(io_v, acc.at[pl.ds(t0 + j * io_rows, io_rows)])
      plsc.subcore_barrier()

      def fire(g, b):
        r0 = tile_row0 + g * _K
        pltpu.sync_copy(comb4.at[pl.ds(q * rows_total + r0, _K)],
                        idx_v.at[b])
        for j in range(_K):
          pltpu.async_copy(feat4.at[idx_v.at[b].at[j, 0]],
                           rows_v.at[b].at[j], gsem[b])

      def wait_gathers(b):
        for j in range(_K):
          pltpu.make_async_copy(zdrain, rows_v.at[b].at[j], gsem[b]).wait()

      fire(0, 0)

      def outer(go, carry):
        for b in range(2):
          g = 2 * go + b

          @pl.when(g < groups - 1)
          def _():
            fire(g + 1, 1 - b)

          wait_gathers(b)
          for j in range(_K):
            pltpu.sync_copy(rows_v.at[b].at[j],
                            acc.at[idx_v.at[b].at[j, 1]], add=True)
        return carry

      lax.fori_loop(0, groups // 2, outer, 0)
      plsc.subcore_barrier()
      for j in range(7):
        pltpu.sync_copy(acc.at[pl.ds(t0 + j * io_rows, io_rows)], io_v)
        pltpu.sync_copy(
            io_v, out.at[pl.ds(q * n_pad + t0 + j * io_rows, io_rows)])
      plsc.subcore_barrier()

  return segsum


def _make_degree(n_pad, rows_total):
  rt = rows_total // (_NC * _NS)
  groups = rt // _K
  t_rows = n_pad // _NS
  io_rows = t_rows // 7
  mesh = plsc.VectorSubcoreMesh(
      core_axis_name="c", subcore_axis_name="s",
      num_cores=_NC, num_subcores=_NS)

  @functools.partial(
      pl.kernel,
      out_type=jax.ShapeDtypeStruct((_NC * n_pad, _QW), jnp.float32),
      mesh=mesh,
      scratch_types=[
          pltpu.VMEM((rt, _LANES), jnp.int32),
          pltpu.VMEM((_LANES, _QW), jnp.float32),
          pltpu.VMEM((io_rows, _QW), jnp.float32),
          pltpu.VMEM_SHARED((n_pad, _QW), jnp.float32),
          pltpu.SemaphoreType.DMA,
      ],
      compiler_params=pltpu.CompilerParams(use_tc_tiling_on_sc=False),
  )
  def degree(dst, ones_hbm, zeros_hbm, out, dst_v, ones_v, io_v, acc, sem):
    c = lax.axis_index("c")
    s = lax.axis_index("s")
    pltpu.sync_copy(ones_hbm, ones_v)
    pltpu.sync_copy(zeros_hbm, io_v)
    t0 = s * t_rows
    for j in range(7):
      pltpu.sync_copy(io_v, acc.at[pl.ds(t0 + j * io_rows, io_rows)])
    plsc.subcore_barrier()

    wid = c * _NS + s
    pltpu.sync_copy(dst.at[pl.ds(wid * rt, rt)], dst_v)

    def group(go, carry):
      for j in range(_K):
        pltpu.async_copy(ones_v, acc.at[dst_v.at[go * _K + j]], sem,
                         add=True)
      return carry

    lax.fori_loop(0, groups, group, 0)

    def drain(go, carry):
      for j in range(_K):
        pltpu.make_async_copy(zeros_hbm.at[pl.ds(0, _LANES)], ones_v,
                              sem).wait()
      return carry

    lax.fori_loop(0, groups, drain, 0)
    plsc.subcore_barrier()
    for j in range(7):
      pltpu.sync_copy(acc.at[pl.ds(t0 + j * io_rows, io_rows)], io_v)
      pltpu.sync_copy(
          io_v, out.at[pl.ds(c * n_pad + t0 + j * io_rows, io_rows)])

  return degree


def _featgen(x_pad, w_pad, b, n_pad):
  nb = n_pad // _BN

  def body(x_ref, w_ref, b_ref, o_ref):
    y = jnp.dot(x_ref[...], w_ref[...], preferred_element_type=jnp.float32)
    y = jnp.maximum(y + b_ref[...], 0.0)
    for q in range(_NQ):
      o_ref[q] = y[:, q * _QW:(q + 1) * _QW]

  return pl.pallas_call(
      body,
      grid=(nb,),
      in_specs=[
          pl.BlockSpec((_BN, 8), lambda i: (i, 0)),
          pl.BlockSpec((8, 64), lambda i: (0, 0)),
          pl.BlockSpec((1, 64), lambda i: (0, 0)),
      ],
      out_specs=pl.BlockSpec((_NQ, _BN, _QW), lambda i: (0, i, 0)),
      out_shape=jax.ShapeDtypeStruct((_NQ, n_pad, _QW), jnp.float32),
  )(x_pad, w_pad, b)


def _router_update(fr, st, sc, deg8, wr, br, n_pad):
  nb = n_pad // _BN

  def body(fr_ref, st_ref, sc_ref, deg_ref, w_ref, b_ref, o_ref):
    inv = 1.0 / jnp.maximum(deg_ref[:, 0:1], 1.0)
    y = b_ref[...]
    for q in range(_NQ):
      y = y + jnp.dot(st_ref[q] * inv, w_ref[q * _QW:(q + 1) * _QW, :],
                      preferred_element_type=jnp.float32)
      y = y + jnp.dot(sc_ref[q], w_ref[64 + q * _QW:64 + (q + 1) * _QW, :],
                      preferred_element_type=jnp.float32)
    y = jnp.maximum(y, 0.0)
    for q in range(_NQ):
      o_ref[q] = fr_ref[q] + y[:, q * _QW:(q + 1) * _QW]

  blk = pl.BlockSpec((_NQ, _BN, _QW), lambda i: (0, i, 0))
  return pl.pallas_call(
      body,
      grid=(nb,),
      in_specs=[
          blk, blk, blk,
          pl.BlockSpec((_BN, _QW), lambda i: (i, 0)),
          pl.BlockSpec((128, 64), lambda i: (0, 0)),
          pl.BlockSpec((1, 64), lambda i: (0, 0)),
      ],
      out_specs=blk,
      out_shape=jax.ShapeDtypeStruct((_NQ, n_pad, _QW), jnp.float32),
  )(fr, st, sc, deg8, wr, br)


def _packet_update(fp, sp, wp, bp, n_pad):
  nb = n_pad // _BN

  def body(fp_ref, sp_ref, w_ref, b_ref, o_ref):
    y = b_ref[...]
    for q in range(_NQ):
      y = y + jnp.dot(sp_ref[q], w_ref[q * _QW:(q + 1) * _QW, :],
                      preferred_element_type=jnp.float32)
    y = jnp.maximum(y, 0.0)
    for q in range(_NQ):
      o_ref[q] = fp_ref[q] + y[:, q * _QW:(q + 1) * _QW]

  blk = pl.BlockSpec((_NQ, _BN, _QW), lambda i: (0, i, 0))
  return pl.pallas_call(
      body,
      grid=(nb,),
      in_specs=[
          blk, blk,
          pl.BlockSpec((64, 64), lambda i: (0, 0)),
          pl.BlockSpec((1, 64), lambda i: (0, 0)),
      ],
      out_specs=blk,
      out_shape=jax.ShapeDtypeStruct((_NQ, n_pad, _QW), jnp.float32),
  )(fp, sp, wp, bp)


def _readout(fr, fp, n_r, n_p, w1, b1, w2, b2, w3p, b3p, n_pad):
  nb = n_pad // _BN

  def body(fr_ref, fp_ref, w1_ref, b1_ref, w2_ref, b2_ref, w3_ref, b3_ref,
           o_ref, acc_ref):
    i = pl.program_id(0)

    @pl.when(i == 0)
    def _():
      acc_ref[...] = jnp.zeros_like(acc_ref)

    rows = i * _BN + lax.broadcasted_iota(jnp.int32, (_BN, 1), 0)
    mp = jnp.where(rows < n_p, 1.0, 0.0)
    mr = jnp.where(rows < n_r, 1.0, 0.0)
    for q in range(_NQ):
      acc_ref[:, q * _QW:(q + 1) * _QW] += jnp.sum(
          fp_ref[q] * mp, axis=0, keepdims=True)
      acc_ref[:, 64 + q * _QW:64 + (q + 1) * _QW] += jnp.sum(
          fr_ref[q] * mr, axis=0, keepdims=True)

    @pl.when(i == nb - 1)
    def _():
      scale = jnp.concatenate(
          [jnp.full((1, 64), 1.0 / n_p, jnp.float32),
           jnp.full((1, 64), 1.0 / n_r, jnp.float32)], axis=1)
      emb = acc_ref[...] * scale
      h = jnp.maximum(
          jnp.dot(emb, w1_ref[...], preferred_element_type=jnp.float32)
          + b1_ref[...], 0.0)
      h = jnp.maximum(
          jnp.dot(h, w2_ref[...], preferred_element_type=jnp.float32)
          + b2_ref[...], 0.0)
      y = jnp.dot(h, w3_ref[...], preferred_element_type=jnp.float32) \
          + b3_ref[...]
      o_ref[...] = jnp.broadcast_to(y, (8, 128))

  blk = pl.BlockSpec((_NQ, _BN, _QW), lambda i: (0, i, 0))
  full = lambda r, c: pl.BlockSpec((r, c), lambda i: (0, 0))
  return pl.pallas_call(
      body,
      grid=(nb,),
      in_specs=[
          blk, blk,
          full(128, 64), full(1, 64),
          full(64, 64), full(1, 64),
          full(64, 128), full(1, 128),
      ],
      out_specs=pl.BlockSpec((8, 128), lambda i: (0, 0)),
      out_shape=jax.ShapeDtypeStruct((8, 128), jnp.float32),
      scratch_shapes=[pltpu.VMEM((1, 128), jnp.float32)],
  )(fr, fp, w1, b1, w2, b2, w3p, b3p)


def kernel(router_embed, packet_embed, pass_edge_index, transfer_edge_index,
           connect_edge_index, W_node, b_node, W_hyper, b_hyper,
           c1_Wr, c1_br, c1_Wp, c1_bp, c2_Wr, c2_br, c2_Wp, c2_bp,
           h_W1, h_b1, h_W2, h_b2, h_W3, h_b3):
  n_r = router_embed.shape[0]
  n_p = packet_embed.shape[0]
  n = max(n_r, n_p)
  n_pad = _ceil_to(n + 1, math.lcm(_NS * 7 * 8, _BN))
  e = pass_edge_index.shape[1]
  rows_total = _ceil_to((e + _LANES - 1) // _LANES, 2 * _NS * _K)
  e_pad = rows_total * _LANES

  def prep_edges(ei):
    src = ei[0].astype(jnp.int32)
    dst = ei[1].astype(jnp.int32)
    src = jnp.pad(src, (0, e_pad - e)).reshape(rows_total, 1, _LANES)
    dst = jnp.pad(dst, (0, e_pad - e), constant_values=n)
    dst = dst.reshape(rows_total, 1, _LANES)
    comb4 = jnp.concatenate(
        [jnp.concatenate([src + q * n_pad, dst], axis=1)
         for q in range(_NQ)], axis=0)
    return comb4, dst.reshape(rows_total, _LANES)

  pass_comb4, pass_dst = prep_edges(pass_edge_index)
  tr_comb4, tr_dst = prep_edges(transfer_edge_index)
  co_comb4, co_dst = prep_edges(connect_edge_index)

  segsum = _make_segsum(n_pad, rows_total)
  degree = _make_degree(n_pad, rows_total)

  zeros16 = jnp.zeros((n_pad // _NS // 7, _QW), jnp.float32)
  ones16 = jnp.ones((_LANES, _QW), jnp.float32)

  deg2 = degree(tr_dst, ones16, zeros16)
  deg8 = deg2[:n_pad] + deg2[n_pad:]

  re_pad = jnp.pad(router_embed, ((0, n_pad - n_r), (0, 8 - 5)))
  pe_pad = jnp.pad(packet_embed, ((0, n_pad - n_p), (0, 8 - 2)))
  wn_pad = jnp.pad(W_node, ((0, 8 - 5), (0, 0)))
  wh_pad = jnp.pad(W_hyper, ((0, 8 - 2), (0, 0)))
  fr = _featgen(re_pad, wn_pad, b_node.reshape(1, 64), n_pad)
  fp = _featgen(pe_pad, wh_pad, b_hyper.reshape(1, 64), n_pad)

  for wr, br, wp, bp in ((c1_Wr, c1_br, c1_Wp, c1_bp),
                         (c2_Wr, c2_br, c2_Wp, c2_bp)):
    fr4 = fr.reshape(_NQ * n_pad, _QW)
    fp4 = fp.reshape(_NQ * n_pad, _QW)
    st = segsum(fp4, tr_comb4, zeros16).reshape(_NQ, n_pad, _QW)
    sc = segsum(fr4, co_comb4, zeros16).reshape(_NQ, n_pad, _QW)
    sp = segsum(fr4, pass_comb4, zeros16).reshape(_NQ, n_pad, _QW)
    fr_new = _router_update(fr, st, sc, deg8, wr, br.reshape(1, 64), n_pad)
    fp_new = _packet_update(fp, sp, wp, bp.reshape(1, 64), n_pad)
    fr, fp = fr_new, fp_new

  w3p = jnp.pad(h_W3, ((0, 0), (0, 128 - 2)))
  b3p = jnp.pad(h_b3, (0, 128 - 2)).reshape(1, 128)
  out = _readout(fr, fp, n_r, n_p, h_W1, h_b1.reshape(1, 64),
                 h_W2, h_b2.reshape(1, 64), w3p, b3p, n_pad)
  return out[0:1, 0:2]


# speedup vs baseline: 1.7106x; 1.0120x over previous
import functools
import math

import jax
import jax.numpy as jnp
from jax import lax
from jax.experimental import pallas as pl
from jax.experimental.pallas import tpu as pltpu
from jax.experimental.pallas import tpu_sc as plsc

_NC = 2
_NS = 16
_NQ = 4
_QW = 16
_LANES = 128
_K = 8

_BN = 1024


def _ceil_to(x, m):
  return (x + m - 1) // m * m


def _make_segsum(n_pad, rows_total):
  rt = rows_total // _NS
  groups = rt // _K
  t_rows = n_pad // _NS
  io_rows = t_rows // 7
  mesh = plsc.VectorSubcoreMesh(
      core_axis_name="c", subcore_axis_name="s",
      num_cores=_NC, num_subcores=_NS)

  @functools.partial(
      pl.kernel,
      out_type=jax.ShapeDtypeStruct((_NQ * n_pad, _QW), jnp.float32),
      mesh=mesh,
      scratch_types=[
          pltpu.VMEM((2, _K, 2, _LANES), jnp.int32),
          pltpu.VMEM((2, _K, _LANES, _QW), jnp.float32),
          pltpu.VMEM((io_rows, _QW), jnp.float32),
          pltpu.VMEM_SHARED((n_pad, _QW), jnp.float32),
          pltpu.SemaphoreType.DMA,
          pltpu.SemaphoreType.DMA,
          pltpu.SemaphoreType.DMA,
          pltpu.SemaphoreType.DMA,
      ],
      compiler_params=pltpu.CompilerParams(use_tc_tiling_on_sc=False),
  )
  def segsum(feat4, comb4, zeros_hbm, out, idx_v, rows_v, io_v,
             acc, gsem0, gsem1, ssem0, ssem1):
    c = lax.axis_index("c")
    s = lax.axis_index("s")
    gsem = (gsem0, gsem1)
    ssem = (ssem0, ssem1)
    zdrain = zeros_hbm.at[pl.ds(0, _LANES)]
    t0 = s * t_rows
    tile_row0 = s * rt

    for q_local in range(2):
      q = c * 2 + q_local
      pltpu.sync_copy(zeros_hbm, io_v)
      for j in range(7):
        pltpu.sync_copy(io_v, acc.at[pl.ds(t0 + j * io_rows, io_rows)])
      plsc.subcore_barrier()

      def fire(g, b):
        r0 = tile_row0 + g * _K
        pltpu.sync_copy(comb4.at[pl.ds(q * rows_total + r0, _K)],
                        idx_v.at[b])
        for j in range(_K):
          pltpu.async_copy(feat4.at[idx_v.at[b].at[j, 0]],
                           rows_v.at[b].at[j], gsem[b])

      def wait_gathers(b):
        for j in range(_K):
          pltpu.make_async_copy(zdrain, rows_v.at[b].at[j], gsem[b]).wait()

      fire(0, 0)

      def outer(go, carry):
        for b in range(2):
          g = 2 * go + b

          @pl.when(g < groups - 1)
          def _():
            fire(g + 1, 1 - b)

          wait_gathers(b)
          hs = []
          for j in range(_K):
            hs.append(pltpu.async_copy(rows_v.at[b].at[j],
                                       acc.at[idx_v.at[b].at[j, 1]],
                                       ssem[b], add=True))
            if j >= 1:
              hs[j - 1].wait()
          hs[_K - 1].wait()
        return carry

      lax.fori_loop(0, groups // 2, outer, 0)
      plsc.subcore_barrier()
      for j in range(7):
        pltpu.sync_copy(acc.at[pl.ds(t0 + j * io_rows, io_rows)], io_v)
        pltpu.sync_copy(
            io_v, out.at[pl.ds(q * n_pad + t0 + j * io_rows, io_rows)])
      plsc.subcore_barrier()

  return segsum


def _make_degree(n_pad, rows_total):
  rt = rows_total // (_NC * _NS)
  groups = rt // _K
  t_rows = n_pad // _NS
  io_rows = t_rows // 7
  mesh = plsc.VectorSubcoreMesh(
      core_axis_name="c", subcore_axis_name="s",
      num_cores=_NC, num_subcores=_NS)

  @functools.partial(
      pl.kernel,
      out_type=jax.ShapeDtypeStruct((_NC * n_pad, _QW), jnp.float32),
      mesh=mesh,
      scratch_types=[
          pltpu.VMEM((rt, _LANES), jnp.int32),
          pltpu.VMEM((_LANES, _QW), jnp.float32),
          pltpu.VMEM((io_rows, _QW), jnp.float32),
          pltpu.VMEM_SHARED((n_pad, _QW), jnp.float32),
          pltpu.SemaphoreType.DMA,
      ],
      compiler_params=pltpu.CompilerParams(use_tc_tiling_on_sc=False),
  )
  def degree(dst, ones_hbm, zeros_hbm, out, dst_v, ones_v, io_v, acc, sem):
    c = lax.axis_index("c")
    s = lax.axis_index("s")
    pltpu.sync_copy(ones_hbm, ones_v)
    pltpu.sync_copy(zeros_hbm, io_v)
    t0 = s * t_rows
    for j in range(7):
      pltpu.sync_copy(io_v, acc.at[pl.ds(t0 + j * io_rows, io_rows)])
    plsc.subcore_barrier()

    wid = c * _NS + s
    pltpu.sync_copy(dst.at[pl.ds(wid * rt, rt)], dst_v)

    def group(go, carry):
      for j in range(_K):
        pltpu.async_copy(ones_v, acc.at[dst_v.at[go * _K + j]], sem,
                         add=True)
      return carry

    lax.fori_loop(0, groups, group, 0)

    def drain(go, carry):
      for j in range(_K):
        pltpu.make_async_copy(zeros_hbm.at[pl.ds(0, _LANES)], ones_v,
                              sem).wait()
      return carry

    lax.fori_loop(0, groups, drain, 0)
    plsc.subcore_barrier()
    for j in range(7):
      pltpu.sync_copy(acc.at[pl.ds(t0 + j * io_rows, io_rows)], io_v)
      pltpu.sync_copy(
          io_v, out.at[pl.ds(c * n_pad + t0 + j * io_rows, io_rows)])

  return degree


def _featgen(x_pad, w_pad, b, n_pad):
  nb = n_pad // _BN

  def body(x_ref, w_ref, b_ref, o_ref):
    y = jnp.dot(x_ref[...], w_ref[...], preferred_element_type=jnp.float32)
    y = jnp.maximum(y + b_ref[...], 0.0)
    for q in range(_NQ):
      o_ref[q] = y[:, q * _QW:(q + 1) * _QW]

  return pl.pallas_call(
      body,
      grid=(nb,),
      in_specs=[
          pl.BlockSpec((_BN, 8), lambda i: (i, 0)),
          pl.BlockSpec((8, 64), lambda i: (0, 0)),
          pl.BlockSpec((1, 64), lambda i: (0, 0)),
      ],
      out_specs=pl.BlockSpec((_NQ, _BN, _QW), lambda i: (0, i, 0)),
      out_shape=jax.ShapeDtypeStruct((_NQ, n_pad, _QW), jnp.float32),
  )(x_pad, w_pad, b)


def _router_update(fr, st, sc, deg8, wr, br, n_pad):
  nb = n_pad // _BN

  def body(fr_ref, st_ref, sc_ref, deg_ref, w_ref, b_ref, o_ref):
    inv = 1.0 / jnp.maximum(deg_ref[:, 0:1], 1.0)
    y = b_ref[...]
    for q in range(_NQ):
      y = y + jnp.dot(st_ref[q] * inv, w_ref[q * _QW:(q + 1) * _QW, :],
                      preferred_element_type=jnp.float32)
      y = y + jnp.dot(sc_ref[q], w_ref[64 + q * _QW:64 + (q + 1) * _QW, :],
                      preferred_element_type=jnp.float32)
    y = jnp.maximum(y, 0.0)
    for q in range(_NQ):
      o_ref[q] = fr_ref[q] + y[:, q * _QW:(q + 1) * _QW]

  blk = pl.BlockSpec((_NQ, _BN, _QW), lambda i: (0, i, 0))
  return pl.pallas_call(
      body,
      grid=(nb,),
      in_specs=[
          blk, blk, blk,
          pl.BlockSpec((_BN, _QW), lambda i: (i, 0)),
          pl.BlockSpec((128, 64), lambda i: (0, 0)),
          pl.BlockSpec((1, 64), lambda i: (0, 0)),
      ],
      out_specs=blk,
      out_shape=jax.ShapeDtypeStruct((_NQ, n_pad, _QW), jnp.float32),
  )(fr, st, sc, deg8, wr, br)


def _packet_update(fp, sp, wp, bp, n_pad):
  nb = n_pad // _BN

  def body(fp_ref, sp_ref, w_ref, b_ref, o_ref):
    y = b_ref[...]
    for q in range(_NQ):
      y = y + jnp.dot(sp_ref[q], w_ref[q * _QW:(q + 1) * _QW, :],
                      preferred_element_type=jnp.float32)
    y = jnp.maximum(y, 0.0)
    for q in range(_NQ):
      o_ref[q] = fp_ref[q] + y[:, q * _QW:(q + 1) * _QW]

  blk = pl.BlockSpec((_NQ, _BN, _QW), lambda i: (0, i, 0))
  return pl.pallas_call(
      body,
      grid=(nb,),
      in_specs=[
          blk, blk,
          pl.BlockSpec((64, 64), lambda i: (0, 0)),
          pl.BlockSpec((1, 64), lambda i: (0, 0)),
      ],
      out_specs=blk,
      out_shape=jax.ShapeDtypeStruct((_NQ, n_pad, _QW), jnp.float32),
  )(fp, sp, wp, bp)


def _readout(fr, fp, n_r, n_p, w1, b1, w2, b2, w3p, b3p, n_pad):
  nb = n_pad // _BN

  def body(fr_ref, fp_ref, w1_ref, b1_ref, w2_ref, b2_ref, w3_ref, b3_ref,
           o_ref, acc_ref):
    i = pl.program_id(0)

    @pl.when(i == 0)
    def _():
      acc_ref[...] = jnp.zeros_like(acc_ref)

    rows = i * _BN + lax.broadcasted_iota(jnp.int32, (_BN, 1), 0)
    mp = jnp.where(rows < n_p, 1.0, 0.0)
    mr = jnp.where(rows < n_r, 1.0, 0.0)
    for q in range(_NQ):
      acc_ref[:, q * _QW:(q + 1) * _QW] += jnp.sum(
          fp_ref[q] * mp, axis=0, keepdims=True)
      acc_ref[:, 64 + q * _QW:64 + (q + 1) * _QW] += jnp.sum(
          fr_ref[q] * mr, axis=0, keepdims=True)

    @pl.when(i == nb - 1)
    def _():
      scale = jnp.concatenate(
          [jnp.full((1, 64), 1.0 / n_p, jnp.float32),
           jnp.full((1, 64), 1.0 / n_r, jnp.float32)], axis=1)
      emb = acc_ref[...] * scale
      h = jnp.maximum(
          jnp.dot(emb, w1_ref[...], preferred_element_type=jnp.float32)
          + b1_ref[...], 0.0)
      h = jnp.maximum(
          jnp.dot(h, w2_ref[...], preferred_element_type=jnp.float32)
          + b2_ref[...], 0.0)
      y = jnp.dot(h, w3_ref[...], preferred_element_type=jnp.float32) \
          + b3_ref[...]
      o_ref[...] = jnp.broadcast_to(y, (8, 128))

  blk = pl.BlockSpec((_NQ, _BN, _QW), lambda i: (0, i, 0))
  full = lambda r, c: pl.BlockSpec((r, c), lambda i: (0, 0))
  return pl.pallas_call(
      body,
      grid=(nb,),
      in_specs=[
          blk, blk,
          full(128, 64), full(1, 64),
          full(64, 64), full(1, 64),
          full(64, 128), full(1, 128),
      ],
      out_specs=pl.BlockSpec((8, 128), lambda i: (0, 0)),
      out_shape=jax.ShapeDtypeStruct((8, 128), jnp.float32),
      scratch_shapes=[pltpu.VMEM((1, 128), jnp.float32)],
  )(fr, fp, w1, b1, w2, b2, w3p, b3p)


def kernel(router_embed, packet_embed, pass_edge_index, transfer_edge_index,
           connect_edge_index, W_node, b_node, W_hyper, b_hyper,
           c1_Wr, c1_br, c1_Wp, c1_bp, c2_Wr, c2_br, c2_Wp, c2_bp,
           h_W1, h_b1, h_W2, h_b2, h_W3, h_b3):
  n_r = router_embed.shape[0]
  n_p = packet_embed.shape[0]
  n = max(n_r, n_p)
  n_pad = _ceil_to(n + 1, math.lcm(_NS * 7 * 8, _BN))
  e = pass_edge_index.shape[1]
  rows_total = _ceil_to((e + _LANES - 1) // _LANES, 2 * _NS * _K)
  e_pad = rows_total * _LANES

  def prep_edges(ei):
    src = ei[0].astype(jnp.int32)
    dst = ei[1].astype(jnp.int32)
    src = jnp.pad(src, (0, e_pad - e)).reshape(rows_total, 1, _LANES)
    dst = jnp.pad(dst, (0, e_pad - e), constant_values=n)
    dst = dst.reshape(rows_total, 1, _LANES)
    comb4 = jnp.concatenate(
        [jnp.concatenate([src + q * n_pad, dst], axis=1)
         for q in range(_NQ)], axis=0)
    return comb4, dst.reshape(rows_total, _LANES)

  pass_comb4, pass_dst = prep_edges(pass_edge_index)
  tr_comb4, tr_dst = prep_edges(transfer_edge_index)
  co_comb4, co_dst = prep_edges(connect_edge_index)

  segsum = _make_segsum(n_pad, rows_total)
  degree = _make_degree(n_pad, rows_total)

  zeros16 = jnp.zeros((n_pad // _NS // 7, _QW), jnp.float32)
  ones16 = jnp.ones((_LANES, _QW), jnp.float32)

  deg2 = degree(tr_dst, ones16, zeros16)
  deg8 = deg2[:n_pad] + deg2[n_pad:]

  re_pad = jnp.pad(router_embed, ((0, n_pad - n_r), (0, 8 - 5)))
  pe_pad = jnp.pad(packet_embed, ((0, n_pad - n_p), (0, 8 - 2)))
  wn_pad = jnp.pad(W_node, ((0, 8 - 5), (0, 0)))
  wh_pad = jnp.pad(W_hyper, ((0, 8 - 2), (0, 0)))
  fr = _featgen(re_pad, wn_pad, b_node.reshape(1, 64), n_pad)
  fp = _featgen(pe_pad, wh_pad, b_hyper.reshape(1, 64), n_pad)

  for wr, br, wp, bp in ((c1_Wr, c1_br, c1_Wp, c1_bp),
                         (c2_Wr, c2_br, c2_Wp, c2_bp)):
    fr4 = fr.reshape(_NQ * n_pad, _QW)
    fp4 = fp.reshape(_NQ * n_pad, _QW)
    st = segsum(fp4, tr_comb4, zeros16).reshape(_NQ, n_pad, _QW)
    sc = segsum(fr4, co_comb4, zeros16).reshape(_NQ, n_pad, _QW)
    sp = segsum(fr4, pass_comb4, zeros16).reshape(_NQ, n_pad, _QW)
    fr_new = _router_update(fr, st, sc, deg8, wr, br.reshape(1, 64), n_pad)
    fp_new = _packet_update(fp, sp, wp, bp.reshape(1, 64), n_pad)
    fr, fp = fr_new, fp_new

  w3p = jnp.pad(h_W3, ((0, 0), (0, 128 - 2)))
  b3p = jnp.pad(h_b3, (0, 128 - 2)).reshape(1, 128)
  out = _readout(fr, fp, n_r, n_p, h_W1, h_b1.reshape(1, 64),
                 h_W2, h_b2.reshape(1, 64), w3p, b3p, n_pad)
  return out[0:1, 0:2]


# speedup vs baseline: 3.0250x; 1.7684x over previous
import functools
import math

import jax
import jax.numpy as jnp
from jax import lax
from jax.experimental import pallas as pl
from jax.experimental.pallas import tpu as pltpu
from jax.experimental.pallas import tpu_sc as plsc

_NC = 2
_NS = 16
_NH = 2
_HW = 32
_LANES = 128
_K = 8

_BN = 1024


def _ceil_to(x, m):
  return (x + m - 1) // m * m


def _make_segsum(n_pad, rows_total):
  rt = rows_total // _NS
  groups = rt // _K
  t_rows = n_pad // _NS
  io_rows = t_rows // 7
  mesh = plsc.VectorSubcoreMesh(
      core_axis_name="c", subcore_axis_name="s",
      num_cores=_NC, num_subcores=_NS)

  @functools.partial(
      pl.kernel,
      out_type=jax.ShapeDtypeStruct((_NH * n_pad, _HW), jnp.bfloat16),
      mesh=mesh,
      scratch_types=[
          pltpu.VMEM((2, _K, 2, _LANES), jnp.int32),
          pltpu.VMEM((2, _K, _LANES, _HW), jnp.bfloat16),
          pltpu.VMEM((io_rows, _HW), jnp.bfloat16),
          pltpu.VMEM_SHARED((n_pad, _HW), jnp.bfloat16),
          pltpu.SemaphoreType.DMA,
          pltpu.SemaphoreType.DMA,
          pltpu.SemaphoreType.DMA,
      ],
      compiler_params=pltpu.CompilerParams(use_tc_tiling_on_sc=False),
  )
  def segsum(table2, comb2, zeros_hbm, out, idx_v, rows_v, io_v,
             acc, gsem0, gsem1, ssem):
    c = lax.axis_index("c")
    s = lax.axis_index("s")
    gsem = (gsem0, gsem1)
    zdrain = zeros_hbm.at[pl.ds(0, _LANES)]
    t0 = s * t_rows
    tile_row0 = s * rt

    pltpu.sync_copy(zeros_hbm, io_v)
    for j in range(7):
      pltpu.sync_copy(io_v, acc.at[pl.ds(t0 + j * io_rows, io_rows)])
    plsc.subcore_barrier()

    def fire(g, b):
      r0 = tile_row0 + g * _K
      pltpu.sync_copy(comb2.at[pl.ds(c * rows_total + r0, _K)], idx_v.at[b])
      for j in range(_K):
        pltpu.async_copy(table2.at[idx_v.at[b].at[j, 0]],
                         rows_v.at[b].at[j], gsem[b])

    def wait_gathers(b):
      for j in range(_K):
        pltpu.make_async_copy(zdrain, rows_v.at[b].at[j], gsem[b]).wait()

    fire(0, 0)

    def outer(go, carry):
      for b in range(2):
        g = 2 * go + b

        @pl.when(g < groups - 1)
        def _():
          fire(g + 1, 1 - b)

        wait_gathers(b)
        hs = []
        for j in range(_K):
          hs.append(pltpu.async_copy(rows_v.at[b].at[j],
                                     acc.at[idx_v.at[b].at[j, 1]],
                                     ssem, add=True))
          if j >= 1:
            hs[j - 1].wait()
        hs[_K - 1].wait()
      return carry

    lax.fori_loop(0, groups // 2, outer, 0)
    plsc.subcore_barrier()
    for j in range(7):
      pltpu.sync_copy(acc.at[pl.ds(t0 + j * io_rows, io_rows)], io_v)
      pltpu.sync_copy(
          io_v, out.at[pl.ds(c * n_pad + t0 + j * io_rows, io_rows)])

  return segsum


def _make_degree(n_pad, rows_total):
  rt = rows_total // (_NC * _NS)
  groups = rt // _K
  t_rows = n_pad // _NS
  io_rows = t_rows // 7
  mesh = plsc.VectorSubcoreMesh(
      core_axis_name="c", subcore_axis_name="s",
      num_cores=_NC, num_subcores=_NS)

  @functools.partial(
      pl.kernel,
      out_type=jax.ShapeDtypeStruct((_NC * n_pad, 16), jnp.float32),
      mesh=mesh,
      scratch_types=[
          pltpu.VMEM((rt, _LANES), jnp.int32),
          pltpu.VMEM((_LANES, 16), jnp.float32),
          pltpu.VMEM((io_rows, 16), jnp.float32),
          pltpu.VMEM_SHARED((n_pad, 16), jnp.float32),
          pltpu.SemaphoreType.DMA,
      ],
      compiler_params=pltpu.CompilerParams(use_tc_tiling_on_sc=False),
  )
  def degree(dst, ones_hbm, zeros_hbm, out, dst_v, ones_v, io_v, acc, sem):
    c = lax.axis_index("c")
    s = lax.axis_index("s")
    pltpu.sync_copy(ones_hbm, ones_v)
    pltpu.sync_copy(zeros_hbm, io_v)
    t0 = s * t_rows
    for j in range(7):
      pltpu.sync_copy(io_v, acc.at[pl.ds(t0 + j * io_rows, io_rows)])
    plsc.subcore_barrier()

    wid = c * _NS + s
    pltpu.sync_copy(dst.at[pl.ds(wid * rt, rt)], dst_v)

    def group(go, carry):
      for j in range(_K):
        pltpu.async_copy(ones_v, acc.at[dst_v.at[go * _K + j]], sem,
                         add=True)
      for j in range(_K):
        pltpu.make_async_copy(zeros_hbm.at[pl.ds(0, _LANES)], ones_v,
                              sem).wait()
      return carry

    lax.fori_loop(0, groups, group, 0)
    plsc.subcore_barrier()
    for j in range(7):
      pltpu.sync_copy(acc.at[pl.ds(t0 + j * io_rows, io_rows)], io_v)
      pltpu.sync_copy(
          io_v, out.at[pl.ds(c * n_pad + t0 + j * io_rows, io_rows)])

  return degree


def _featgen(x_pad, w_pad, b, n_pad):
  nb = n_pad // _BN

  def body(x_ref, w_ref, b_ref, o_ref, t_ref):
    y = jnp.dot(x_ref[...], w_ref[...], preferred_element_type=jnp.float32)
    y = jnp.maximum(y + b_ref[...], 0.0)
    for h in range(_NH):
      o_ref[h] = y[:, h * _HW:(h + 1) * _HW]
      t_ref[h] = y[:, h * _HW:(h + 1) * _HW].astype(jnp.bfloat16)

  blk = pl.BlockSpec((_NH, _BN, _HW), lambda i: (0, i, 0))
  return pl.pallas_call(
      body,
      grid=(nb,),
      in_specs=[
          pl.BlockSpec((_BN, 8), lambda i: (i, 0)),
          pl.BlockSpec((8, 64), lambda i: (0, 0)),
          pl.BlockSpec((1, 64), lambda i: (0, 0)),
      ],
      out_specs=[blk, blk],
      out_shape=[
          jax.ShapeDtypeStruct((_NH, n_pad, _HW), jnp.float32),
          jax.ShapeDtypeStruct((_NH, n_pad, _HW), jnp.bfloat16),
      ],
  )(x_pad, w_pad, b)


def _router_update(fr, st, sc, deg16, wr, br, n_pad):
  nb = n_pad // _BN

  def body(fr_ref, st_ref, sc_ref, deg_ref, w_ref, b_ref, o_ref, t_ref):
    inv = 1.0 / jnp.maximum(deg_ref[:, 0:1], 1.0)
    y = b_ref[...]
    for h in range(_NH):
      y = y + jnp.dot(st_ref[h].astype(jnp.float32) * inv,
                      w_ref[h * _HW:(h + 1) * _HW, :],
                      preferred_element_type=jnp.float32)
      y = y + jnp.dot(sc_ref[h].astype(jnp.float32),
                      w_ref[64 + h * _HW:64 + (h + 1) * _HW, :],
                      preferred_element_type=jnp.float32)
    y = jnp.maximum(y, 0.0)
    for h in range(_NH):
      z = fr_ref[h] + y[:, h * _HW:(h + 1) * _HW]
      o_ref[h] = z
      t_ref[h] = z.astype(jnp.bfloat16)

  blk = pl.BlockSpec((_NH, _BN, _HW), lambda i: (0, i, 0))
  bblk = pl.BlockSpec((_NH, _BN, _HW), lambda i: (0, i, 0))
  return pl.pallas_call(
      body,
      grid=(nb,),
      in_specs=[
          blk, bblk, bblk,
          pl.BlockSpec((_BN, 16), lambda i: (i, 0)),
          pl.BlockSpec((128, 64), lambda i: (0, 0)),
          pl.BlockSpec((1, 64), lambda i: (0, 0)),
      ],
      out_specs=[blk, blk],
      out_shape=[
          jax.ShapeDtypeStruct((_NH, n_pad, _HW), jnp.float32),
          jax.ShapeDtypeStruct((_NH, n_pad, _HW), jnp.bfloat16),
      ],
  )(fr, st, sc, deg16, wr, br)


def _packet_update(fp, sp, wp, bp, n_pad):
  nb = n_pad // _BN

  def body(fp_ref, sp_ref, w_ref, b_ref, o_ref, t_ref):
    y = b_ref[...]
    for h in range(_NH):
      y = y + jnp.dot(sp_ref[h].astype(jnp.float32),
                      w_ref[h * _HW:(h + 1) * _HW, :],
                      preferred_element_type=jnp.float32)
    y = jnp.maximum(y, 0.0)
    for h in range(_NH):
      z = fp_ref[h] + y[:, h * _HW:(h + 1) * _HW]
      o_ref[h] = z
      t_ref[h] = z.astype(jnp.bfloat16)

  blk = pl.BlockSpec((_NH, _BN, _HW), lambda i: (0, i, 0))
  return pl.pallas_call(
      body,
      grid=(nb,),
      in_specs=[
          blk, blk,
          pl.BlockSpec((64, 64), lambda i: (0, 0)),
          pl.BlockSpec((1, 64), lambda i: (0, 0)),
      ],
      out_specs=[blk, blk],
      out_shape=[
          jax.ShapeDtypeStruct((_NH, n_pad, _HW), jnp.float32),
          jax.ShapeDtypeStruct((_NH, n_pad, _HW), jnp.bfloat16),
      ],
  )(fp, sp, wp, bp)


def _readout(fr, fp, n_r, n_p, w1, b1, w2, b2, w3p, b3p, n_pad):
  nb = n_pad // _BN

  def body(fr_ref, fp_ref, w1_ref, b1_ref, w2_ref, b2_ref, w3_ref, b3_ref,
           o_ref, acc_ref):
    i = pl.program_id(0)

    @pl.when(i == 0)
    def _():
      acc_ref[...] = jnp.zeros_like(acc_ref)

    rows = i * _BN + lax.broadcasted_iota(jnp.int32, (_BN, 1), 0)
    mp = jnp.where(rows < n_p, 1.0, 0.0)
    mr = jnp.where(rows < n_r, 1.0, 0.0)
    for h in range(_NH):
      acc_ref[:, h * _HW:(h + 1) * _HW] += jnp.sum(
          fp_ref[h] * mp, axis=0, keepdims=True)
      acc_ref[:, 64 + h * _HW:64 + (h + 1) * _HW] += jnp.sum(
          fr_ref[h] * mr, axis=0, keepdims=True)

    @pl.when(i == nb - 1)
    def _():
      scale = jnp.concatenate(
          [jnp.full((1, 64), 1.0 / n_p, jnp.float32),
           jnp.full((1, 64), 1.0 / n_r, jnp.float32)], axis=1)
      emb = acc_ref[...] * scale
      h = jnp.maximum(
          jnp.dot(emb, w1_ref[...], preferred_element_type=jnp.float32)
          + b1_ref[...], 0.0)
      h = jnp.maximum(
          jnp.dot(h, w2_ref[...], preferred_element_type=jnp.float32)
          + b2_ref[...], 0.0)
      y = jnp.dot(h, w3_ref[...], preferred_element_type=jnp.float32) \
          + b3_ref[...]
      o_ref[...] = jnp.broadcast_to(y, (8, 128))

  blk = pl.BlockSpec((_NH, _BN, _HW), lambda i: (0, i, 0))
  full = lambda r, c: pl.BlockSpec((r, c), lambda i: (0, 0))
  return pl.pallas_call(
      body,
      grid=(nb,),
      in_specs=[
          blk, blk,
          full(128, 64), full(1, 64),
          full(64, 64), full(1, 64),
          full(64, 128), full(1, 128),
      ],
      out_specs=pl.BlockSpec((8, 128), lambda i: (0, 0)),
      out_shape=jax.ShapeDtypeStruct((8, 128), jnp.float32),
      scratch_shapes=[pltpu.VMEM((1, 128), jnp.float32)],
  )(fr, fp, w1, b1, w2, b2, w3p, b3p)


def kernel(router_embed, packet_embed, pass_edge_index, transfer_edge_index,
           connect_edge_index, W_node, b_node, W_hyper, b_hyper,
           c1_Wr, c1_br, c1_Wp, c1_bp, c2_Wr, c2_br, c2_Wp, c2_bp,
           h_W1, h_b1, h_W2, h_b2, h_W3, h_b3):
  n_r = router_embed.shape[0]
  n_p = packet_embed.shape[0]
  n = max(n_r, n_p)
  n_pad = _ceil_to(n + 1, math.lcm(_NS * 7 * 16, _BN))
  e = pass_edge_index.shape[1]
  rows_total = _ceil_to((e + _LANES - 1) // _LANES, 2 * _NS * _K)
  e_pad = rows_total * _LANES

  def prep_edges(ei):
    src = ei[0].astype(jnp.int32)
    dst = ei[1].astype(jnp.int32)
    src = jnp.pad(src, (0, e_pad - e)).reshape(rows_total, 1, _LANES)
    dst = jnp.pad(dst, (0, e_pad - e), constant_values=n)
    dst = dst.reshape(rows_total, 1, _LANES)
    comb2 = jnp.concatenate(
        [jnp.concatenate([src + h * n_pad, dst], axis=1)
         for h in range(_NH)], axis=0)
    return comb2, dst.reshape(rows_total, _LANES)

  pass_comb2, pass_dst = prep_edges(pass_edge_index)
  tr_comb2, tr_dst = prep_edges(transfer_edge_index)
  co_comb2, co_dst = prep_edges(connect_edge_index)

  segsum = _make_segsum(n_pad, rows_total)
  degree = _make_degree(n_pad, rows_total)

  zeros_bf = jnp.zeros((n_pad // _NS // 7, _HW), jnp.bfloat16)
  zeros_f32 = jnp.zeros((n_pad // _NS // 7, 16), jnp.float32)
  ones16 = jnp.ones((_LANES, 16), jnp.float32)

  deg2 = degree(tr_dst, ones16, zeros_f32)
  deg16 = deg2[:n_pad] + deg2[n_pad:]

  re_pad = jnp.pad(router_embed, ((0, n_pad - n_r), (0, 8 - 5)))
  pe_pad = jnp.pad(packet_embed, ((0, n_pad - n_p), (0, 8 - 2)))
  wn_pad = jnp.pad(W_node, ((0, 8 - 5), (0, 0)))
  wh_pad = jnp.pad(W_hyper, ((0, 8 - 2), (0, 0)))
  fr, fr_t = _featgen(re_pad, wn_pad, b_node.reshape(1, 64), n_pad)
  fp, fp_t = _featgen(pe_pad, wh_pad, b_hyper.reshape(1, 64), n_pad)

  for wr, br, wp, bp in ((c1_Wr, c1_br, c1_Wp, c1_bp),
                         (c2_Wr, c2_br, c2_Wp, c2_bp)):
    fr2 = fr_t.reshape(_NH * n_pad, _HW)
    fp2 = fp_t.reshape(_NH * n_pad, _HW)
    st = segsum(fp2, tr_comb2, zeros_bf).reshape(_NH, n_pad, _HW)
    sc = segsum(fr2, co_comb2, zeros_bf).reshape(_NH, n_pad, _HW)
    sp = segsum(fr2, pass_comb2, zeros_bf).reshape(_NH, n_pad, _HW)
    fr, fr_t = _router_update(fr, st, sc, deg16, wr, br.reshape(1, 64),
                              n_pad)
    fp, fp_t = _packet_update(fp, sp, wp, bp.reshape(1, 64), n_pad)

  w3p = jnp.pad(h_W3, ((0, 0), (0, 128 - 2)))
  b3p = jnp.pad(h_b3, (0, 128 - 2)).reshape(1, 128)
  out = _readout(fr, fp, n_r, n_p, h_W1, h_b1.reshape(1, 64),
                 h_W2, h_b2.reshape(1, 64), w3p, b3p, n_pad)
  return out[0:1, 0:2]


# speedup vs baseline: 3.0404x; 1.0051x over previous
import functools
import math

import jax
import jax.numpy as jnp
from jax import lax
from jax.experimental import pallas as pl
from jax.experimental.pallas import tpu as pltpu
from jax.experimental.pallas import tpu_sc as plsc

_NC = 2
_NS = 16
_NH = 2
_HW = 32
_LANES = 128
_K = 8

_BN = 1024


def _ceil_to(x, m):
  return (x + m - 1) // m * m


def _make_segsum(n_pad, rows_total):
  rt = rows_total // _NS
  groups = rt // _K
  t_rows = n_pad // _NS
  io_rows = t_rows // 7
  mesh = plsc.VectorSubcoreMesh(
      core_axis_name="c", subcore_axis_name="s",
      num_cores=_NC, num_subcores=_NS)

  @functools.partial(
      pl.kernel,
      out_type=jax.ShapeDtypeStruct((_NH * n_pad, _HW), jnp.bfloat16),
      mesh=mesh,
      scratch_types=[
          pltpu.VMEM((2, _K, 2, _LANES), jnp.int32),
          pltpu.VMEM((2, _K, _LANES, _HW), jnp.bfloat16),
          pltpu.VMEM((io_rows, _HW), jnp.bfloat16),
          pltpu.VMEM((io_rows, _HW), jnp.bfloat16),
          pltpu.VMEM_SHARED((n_pad, _HW), jnp.bfloat16),
          pltpu.SemaphoreType.DMA,
          pltpu.SemaphoreType.DMA,
          pltpu.SemaphoreType.DMA,
      ],
      compiler_params=pltpu.CompilerParams(use_tc_tiling_on_sc=False),
  )
  def segsum(table2, comb2, zeros_hbm, out, idx_v, rows_v, io_v, io2_v,
             acc, gsem0, gsem1, ssem):
    c = lax.axis_index("c")
    s = lax.axis_index("s")
    gsem = (gsem0, gsem1)
    zdrain = zeros_hbm.at[pl.ds(0, _LANES)]
    t0 = s * t_rows
    tile_row0 = s * rt

    pltpu.sync_copy(zeros_hbm, io_v)
    for j in range(7):
      pltpu.async_copy(io_v, acc.at[pl.ds(t0 + j * io_rows, io_rows)], gsem0)
    for j in range(7):
      pltpu.make_async_copy(zeros_hbm, io_v, gsem0).wait()
    plsc.subcore_barrier()

    def fire(g, b):
      r0 = tile_row0 + g * _K
      pltpu.sync_copy(comb2.at[pl.ds(c * rows_total + r0, _K)], idx_v.at[b])
      for j in range(_K):
        pltpu.async_copy(table2.at[idx_v.at[b].at[j, 0]],
                         rows_v.at[b].at[j], gsem[b])

    def wait_gathers(b):
      for j in range(_K):
        pltpu.make_async_copy(zdrain, rows_v.at[b].at[j], gsem[b]).wait()

    fire(0, 0)

    def outer(go, carry):
      for b in range(2):
        g = 2 * go + b

        @pl.when(g < groups - 1)
        def _():
          fire(g + 1, 1 - b)

        wait_gathers(b)
        hs = []
        for j in range(_K):
          hs.append(pltpu.async_copy(rows_v.at[b].at[j],
                                     acc.at[idx_v.at[b].at[j, 1]],
                                     ssem, add=True))
          if j >= 1:
            hs[j - 1].wait()
        hs[_K - 1].wait()
      return carry

    lax.fori_loop(0, groups // 2, outer, 0)
    plsc.subcore_barrier()
    ios = (io_v, io2_v)
    for j in range(7):
      bb = j % 2
      if j >= 2:
        pltpu.make_async_copy(zeros_hbm, ios[bb], gsem0).wait()
      pltpu.sync_copy(acc.at[pl.ds(t0 + j * io_rows, io_rows)], ios[bb])
      pltpu.async_copy(
          ios[bb], out.at[pl.ds(c * n_pad + t0 + j * io_rows, io_rows)],
          gsem0)
    for j in range(2):
      pltpu.make_async_copy(zeros_hbm, ios[j], gsem0).wait()

  return segsum


def _make_degree(n_pad, rows_total):
  rt = rows_total // (_NC * _NS)
  groups = rt // _K
  t_rows = n_pad // _NS
  io_rows = t_rows // 7
  mesh = plsc.VectorSubcoreMesh(
      core_axis_name="c", subcore_axis_name="s",
      num_cores=_NC, num_subcores=_NS)

  @functools.partial(
      pl.kernel,
      out_type=jax.ShapeDtypeStruct((_NC * n_pad, 16), jnp.float32),
      mesh=mesh,
      scratch_types=[
          pltpu.VMEM((rt, _LANES), jnp.int32),
          pltpu.VMEM((_LANES, 16), jnp.float32),
          pltpu.VMEM((io_rows, 16), jnp.float32),
          pltpu.VMEM_SHARED((n_pad, 16), jnp.float32),
          pltpu.SemaphoreType.DMA,
      ],
      compiler_params=pltpu.CompilerParams(use_tc_tiling_on_sc=False),
  )
  def degree(dst, ones_hbm, zeros_hbm, out, dst_v, ones_v, io_v, acc, sem):
    c = lax.axis_index("c")
    s = lax.axis_index("s")
    pltpu.sync_copy(ones_hbm, ones_v)
    pltpu.sync_copy(zeros_hbm, io_v)
    t0 = s * t_rows
    for j in range(7):
      pltpu.sync_copy(io_v, acc.at[pl.ds(t0 + j * io_rows, io_rows)])
    plsc.subcore_barrier()

    wid = c * _NS + s
    pltpu.sync_copy(dst.at[pl.ds(wid * rt, rt)], dst_v)

    def group(go, carry):
      for j in range(_K):
        pltpu.async_copy(ones_v, acc.at[dst_v.at[go * _K + j]], sem,
                         add=True)
      for j in range(_K):
        pltpu.make_async_copy(zeros_hbm.at[pl.ds(0, _LANES)], ones_v,
                              sem).wait()
      return carry

    lax.fori_loop(0, groups, group, 0)
    plsc.subcore_barrier()
    for j in range(7):
      pltpu.sync_copy(acc.at[pl.ds(t0 + j * io_rows, io_rows)], io_v)
      pltpu.sync_copy(
          io_v, out.at[pl.ds(c * n_pad + t0 + j * io_rows, io_rows)])

  return degree


def _featgen(x_pad, w_pad, b, n_pad):
  nb = n_pad // _BN

  def body(x_ref, w_ref, b_ref, o_ref, t_ref):
    y = jnp.dot(x_ref[...], w_ref[...], preferred_element_type=jnp.float32)
    y = jnp.maximum(y + b_ref[...], 0.0)
    for h in range(_NH):
      o_ref[h] = y[:, h * _HW:(h + 1) * _HW]
      t_ref[h] = y[:, h * _HW:(h + 1) * _HW].astype(jnp.bfloat16)

  blk = pl.BlockSpec((_NH, _BN, _HW), lambda i: (0, i, 0))
  return pl.pallas_call(
      body,
      grid=(nb,),
      in_specs=[
          pl.BlockSpec((_BN, 8), lambda i: (i, 0)),
          pl.BlockSpec((8, 64), lambda i: (0, 0)),
          pl.BlockSpec((1, 64), lambda i: (0, 0)),
      ],
      out_specs=[blk, blk],
      out_shape=[
          jax.ShapeDtypeStruct((_NH, n_pad, _HW), jnp.float32),
          jax.ShapeDtypeStruct((_NH, n_pad, _HW), jnp.bfloat16),
      ],
  )(x_pad, w_pad, b)


def _router_update(fr, st, sc, deg16, wr, br, n_pad):
  nb = n_pad // _BN

  def body(fr_ref, st_ref, sc_ref, deg_ref, w_ref, b_ref, o_ref, t_ref):
    inv = 1.0 / jnp.maximum(deg_ref[:, 0:1], 1.0)
    y = b_ref[...]
    for h in range(_NH):
      y = y + jnp.dot(st_ref[h].astype(jnp.float32) * inv,
                      w_ref[h * _HW:(h + 1) * _HW, :],
                      preferred_element_type=jnp.float32)
      y = y + jnp.dot(sc_ref[h].astype(jnp.float32),
                      w_ref[64 + h * _HW:64 + (h + 1) * _HW, :],
                      preferred_element_type=jnp.float32)
    y = jnp.maximum(y, 0.0)
    for h in range(_NH):
      z = fr_ref[h] + y[:, h * _HW:(h + 1) * _HW]
      o_ref[h] = z
      t_ref[h] = z.astype(jnp.bfloat16)

  blk = pl.BlockSpec((_NH, _BN, _HW), lambda i: (0, i, 0))
  bblk = pl.BlockSpec((_NH, _BN, _HW), lambda i: (0, i, 0))
  return pl.pallas_call(
      body,
      grid=(nb,),
      in_specs=[
          blk, bblk, bblk,
          pl.BlockSpec((_BN, 16), lambda i: (i, 0)),
          pl.BlockSpec((128, 64), lambda i: (0, 0)),
          pl.BlockSpec((1, 64), lambda i: (0, 0)),
      ],
      out_specs=[blk, blk],
      out_shape=[
          jax.ShapeDtypeStruct((_NH, n_pad, _HW), jnp.float32),
          jax.ShapeDtypeStruct((_NH, n_pad, _HW), jnp.bfloat16),
      ],
  )(fr, st, sc, deg16, wr, br)


def _packet_update(fp, sp, wp, bp, n_pad):
  nb = n_pad // _BN

  def body(fp_ref, sp_ref, w_ref, b_ref, o_ref, t_ref):
    y = b_ref[...]
    for h in range(_NH):
      y = y + jnp.dot(sp_ref[h].astype(jnp.float32),
                      w_ref[h * _HW:(h + 1) * _HW, :],
                      preferred_element_type=jnp.float32)
    y = jnp.maximum(y, 0.0)
    for h in range(_NH):
      z = fp_ref[h] + y[:, h * _HW:(h + 1) * _HW]
      o_ref[h] = z
      t_ref[h] = z.astype(jnp.bfloat16)

  blk = pl.BlockSpec((_NH, _BN, _HW), lambda i: (0, i, 0))
  return pl.pallas_call(
      body,
      grid=(nb,),
      in_specs=[
          blk, blk,
          pl.BlockSpec((64, 64), lambda i: (0, 0)),
          pl.BlockSpec((1, 64), lambda i: (0, 0)),
      ],
      out_specs=[blk, blk],
      out_shape=[
          jax.ShapeDtypeStruct((_NH, n_pad, _HW), jnp.float32),
          jax.ShapeDtypeStruct((_NH, n_pad, _HW), jnp.bfloat16),
      ],
  )(fp, sp, wp, bp)


def _readout(fr, fp, n_r, n_p, w1, b1, w2, b2, w3p, b3p, n_pad):
  nb = n_pad // _BN

  def body(fr_ref, fp_ref, w1_ref, b1_ref, w2_ref, b2_ref, w3_ref, b3_ref,
           o_ref, acc_ref):
    i = pl.program_id(0)

    @pl.when(i == 0)
    def _():
      acc_ref[...] = jnp.zeros_like(acc_ref)

    rows = i * _BN + lax.broadcasted_iota(jnp.int32, (_BN, 1), 0)
    mp = jnp.where(rows < n_p, 1.0, 0.0)
    mr = jnp.where(rows < n_r, 1.0, 0.0)
    for h in range(_NH):
      acc_ref[:, h * _HW:(h + 1) * _HW] += jnp.sum(
          fp_ref[h] * mp, axis=0, keepdims=True)
      acc_ref[:, 64 + h * _HW:64 + (h + 1) * _HW] += jnp.sum(
          fr_ref[h] * mr, axis=0, keepdims=True)

    @pl.when(i == nb - 1)
    def _():
      scale = jnp.concatenate(
          [jnp.full((1, 64), 1.0 / n_p, jnp.float32),
           jnp.full((1, 64), 1.0 / n_r, jnp.float32)], axis=1)
      emb = acc_ref[...] * scale
      h = jnp.maximum(
          jnp.dot(emb, w1_ref[...], preferred_element_type=jnp.float32)
          + b1_ref[...], 0.0)
      h = jnp.maximum(
          jnp.dot(h, w2_ref[...], preferred_element_type=jnp.float32)
          + b2_ref[...], 0.0)
      y = jnp.dot(h, w3_ref[...], preferred_element_type=jnp.float32) \
          + b3_ref[...]
      o_ref[...] = jnp.broadcast_to(y, (8, 128))

  blk = pl.BlockSpec((_NH, _BN, _HW), lambda i: (0, i, 0))
  full = lambda r, c: pl.BlockSpec((r, c), lambda i: (0, 0))
  return pl.pallas_call(
      body,
      grid=(nb,),
      in_specs=[
          blk, blk,
          full(128, 64), full(1, 64),
          full(64, 64), full(1, 64),
          full(64, 128), full(1, 128),
      ],
      out_specs=pl.BlockSpec((8, 128), lambda i: (0, 0)),
      out_shape=jax.ShapeDtypeStruct((8, 128), jnp.float32),
      scratch_shapes=[pltpu.VMEM((1, 128), jnp.float32)],
  )(fr, fp, w1, b1, w2, b2, w3p, b3p)


def kernel(router_embed, packet_embed, pass_edge_index, transfer_edge_index,
           connect_edge_index, W_node, b_node, W_hyper, b_hyper,
           c1_Wr, c1_br, c1_Wp, c1_bp, c2_Wr, c2_br, c2_Wp, c2_bp,
           h_W1, h_b1, h_W2, h_b2, h_W3, h_b3):
  n_r = router_embed.shape[0]
  n_p = packet_embed.shape[0]
  n = max(n_r, n_p)
  n_pad = _ceil_to(n + 1, math.lcm(_NS * 7 * 16, _BN))
  e = pass_edge_index.shape[1]
  rows_total = _ceil_to((e + _LANES - 1) // _LANES, 2 * _NS * _K)
  e_pad = rows_total * _LANES

  def prep_edges(ei):
    src = ei[0].astype(jnp.int32)
    dst = ei[1].astype(jnp.int32)
    src = jnp.pad(src, (0, e_pad - e)).reshape(rows_total, 1, _LANES)
    dst = jnp.pad(dst, (0, e_pad - e), constant_values=n)
    dst = dst.reshape(rows_total, 1, _LANES)
    comb2 = jnp.concatenate(
        [jnp.concatenate([src + h * n_pad, dst], axis=1)
         for h in range(_NH)], axis=0)
    return comb2, dst.reshape(rows_total, _LANES)

  pass_comb2, pass_dst = prep_edges(pass_edge_index)
  tr_comb2, tr_dst = prep_edges(transfer_edge_index)
  co_comb2, co_dst = prep_edges(connect_edge_index)

  segsum = _make_segsum(n_pad, rows_total)
  degree = _make_degree(n_pad, rows_total)

  zeros_bf = jnp.zeros((n_pad // _NS // 7, _HW), jnp.bfloat16)
  zeros_f32 = jnp.zeros((n_pad // _NS // 7, 16), jnp.float32)
  ones16 = jnp.ones((_LANES, 16), jnp.float32)

  deg2 = degree(tr_dst, ones16, zeros_f32)
  deg16 = deg2[:n_pad] + deg2[n_pad:]

  re_pad = jnp.pad(router_embed, ((0, n_pad - n_r), (0, 8 - 5)))
  pe_pad = jnp.pad(packet_embed, ((0, n_pad - n_p), (0, 8 - 2)))
  wn_pad = jnp.pad(W_node, ((0, 8 - 5), (0, 0)))
  wh_pad = jnp.pad(W_hyper, ((0, 8 - 2), (0, 0)))
  fr, fr_t = _featgen(re_pad, wn_pad, b_node.reshape(1, 64), n_pad)
  fp, fp_t = _featgen(pe_pad, wh_pad, b_hyper.reshape(1, 64), n_pad)

  for wr, br, wp, bp in ((c1_Wr, c1_br, c1_Wp, c1_bp),
                         (c2_Wr, c2_br, c2_Wp, c2_bp)):
    fr2 = fr_t.reshape(_NH * n_pad, _HW)
    fp2 = fp_t.reshape(_NH * n_pad, _HW)
    st = segsum(fp2, tr_comb2, zeros_bf).reshape(_NH, n_pad, _HW)
    sc = segsum(fr2, co_comb2, zeros_bf).reshape(_NH, n_pad, _HW)
    sp = segsum(fr2, pass_comb2, zeros_bf).reshape(_NH, n_pad, _HW)
    fr, fr_t = _router_update(fr, st, sc, deg16, wr, br.reshape(1, 64),
                              n_pad)
    fp, fp_t = _packet_update(fp, sp, wp, bp.reshape(1, 64), n_pad)

  w3p = jnp.pad(h_W3, ((0, 0), (0, 128 - 2)))
  b3p = jnp.pad(h_b3, (0, 128 - 2)).reshape(1, 128)
  out = _readout(fr, fp, n_r, n_p, h_W1, h_b1.reshape(1, 64),
                 h_W2, h_b2.reshape(1, 64), w3p, b3p, n_pad)
  return out[0:1, 0:2]
